# baseline (speedup 1.0000x reference)
import jax
import jax.numpy as jnp
from jax.experimental import pallas as pl
from jax.experimental.pallas import tpu as pltpu

N = 10000
F_IN = 128
HPAD = 16
BLK = 400
NBLK = N // BLK


def _gcn_body(x_ref, adj_ref, w1_ref, b1_ref, w2_ref, b2_ref, w3_ref, b3_ref,
              out_ref, s1_ref, s2_ref, max_ref):
    p = pl.program_id(0)
    i = pl.program_id(1)

    @pl.when((p == 0) & (i == 0))
    def _():
        s1_ref[...] = jnp.dot(x_ref[...], w1_ref[...],
                              preferred_element_type=jnp.float32)

    @pl.when(p == 0)
    def _():
        t = jnp.dot(adj_ref[...], s1_ref[...],
                    preferred_element_type=jnp.float32)
        h = jnp.maximum(t + b1_ref[...], 0.0)
        s2_ref[pl.ds(i * BLK, BLK), :] = jnp.dot(
            h, w2_ref[...], preferred_element_type=jnp.float32)

    @pl.when(p == 1)
    def _():
        u = jnp.dot(adj_ref[...], s2_ref[...],
                    preferred_element_type=jnp.float32)
        m = jnp.max(u, axis=0, keepdims=True)
        prev = jnp.where(i == 0, jnp.full((1, HPAD), -jnp.inf, jnp.float32),
                         max_ref[0:1, :])
        max_ref[0:1, :] = jnp.maximum(prev, m)

    @pl.when((p == 1) & (i == NBLK - 1))
    def _():
        pooled = max_ref[0:1, :] + b2_ref[...]
        val = jnp.sum(pooled * w3_ref[...]) + b3_ref[0, 0]
        out_ref[...] = jnp.full((8, 128), val, jnp.float32)


def kernel(x, adj, W1, b1, W2, b2, W3, b3):
    w2p = jnp.zeros((HPAD, HPAD), jnp.float32).at[:, :2].set(W2)
    b2p = jnp.zeros((1, HPAD), jnp.float32).at[0, :2].set(b2)
    w3p = jnp.zeros((1, HPAD), jnp.float32).at[0, :2].set(W3[:, 0])
    b1r = b1.reshape(1, HPAD)
    b3r = b3.reshape(1, 1)

    out = pl.pallas_call(
        _gcn_body,
        grid=(2, NBLK),
        in_specs=[
            pl.BlockSpec((N, F_IN), lambda p, i: (0, 0)),
            pl.BlockSpec((BLK, N), lambda p, i: (i, 0)),
            pl.BlockSpec((F_IN, HPAD), lambda p, i: (0, 0)),
            pl.BlockSpec((1, HPAD), lambda p, i: (0, 0)),
            pl.BlockSpec((HPAD, HPAD), lambda p, i: (0, 0)),
            pl.BlockSpec((1, HPAD), lambda p, i: (0, 0)),
            pl.BlockSpec((1, HPAD), lambda p, i: (0, 0)),
            pl.BlockSpec((1, 1), lambda p, i: (0, 0)),
        ],
        out_specs=pl.BlockSpec((8, 128), lambda p, i: (0, 0)),
        out_shape=jax.ShapeDtypeStruct((8, 128), jnp.float32),
        scratch_shapes=[
            pltpu.VMEM((N, HPAD), jnp.float32),
            pltpu.VMEM((N, HPAD), jnp.float32),
            pltpu.VMEM((8, HPAD), jnp.float32),
        ],
    )(x, adj, W1, b1r, w2p, b2p, w3p, b3r)
    return out[0, 0].reshape(1, 1, 1)
